# baseline (speedup 1.0000x reference)
import numpy as np
import jax
import jax.numpy as jnp
from jax.experimental import pallas as pl
from jax.experimental.pallas import tpu as pltpu

_B, _L, _N = 32, 336, 8
_E, _K = 4, 2
_DM, _DF = 128, 256
_PATCHES = (8, 6, 4, 2)
_MA_KERNELS = (4, 8, 12)


def _dgt(a, w):
    return jax.lax.dot_general(a, w, (((1,), (1,)), ((), ())),
                               preferred_element_type=jnp.float32)


def _gating_kernel(x2_ref, dec_ref, sw_ref, gw_ref, gb_ref, gates_ref, loss_ref):
    x = x2_ref[...]

    li = jax.lax.broadcasted_iota(jnp.int32, (_L, _L), 0)
    lp = jax.lax.broadcasted_iota(jnp.int32, (_L, _L), 1)
    means = []
    for k in _MA_KERNELS:
        lo = li - (k - 1) // 2
        hi = li + k // 2
        w = ((lp >= lo) & (lp <= hi)).astype(jnp.float32)
        w = w + jnp.where(lp == 0, jnp.maximum(-lo, 0).astype(jnp.float32), 0.0)
        w = w + jnp.where(lp == _L - 1,
                          jnp.maximum(hi - (_L - 1), 0).astype(jnp.float32), 0.0)
        wk = w * (1.0 / k)
        means.append(_dgt(x, wk))

    lgs = [x * dec_ref[0, k] + dec_ref[1, k] for k in range(len(_MA_KERNELS))]
    m = jnp.maximum(jnp.maximum(lgs[0], lgs[1]), lgs[2])
    es = [jnp.exp(lg - m) for lg in lgs]
    z = es[0] + es[1] + es[2]
    mean = (means[0] * es[0] + means[1] * es[1] + means[2] * es[2]) / z
    new_x = x + mean

    r = jax.lax.broadcasted_iota(jnp.int32, (_B, _B * _N), 0)
    c = jax.lax.broadcasted_iota(jnp.int32, (_B, _B * _N), 1)
    swcol = jnp.zeros((_B, _B * _N), jnp.float32)
    for n in range(_N):
        swcol = swcol + jnp.where((c & (_N - 1)) == n, sw_ref[0, n], 0.0)
    s_mat = jnp.where((c >> 3) == r, swcol, 0.0)
    xg = jax.lax.dot_general(s_mat, new_x, (((1,), (0,)), ((), ())),
                             preferred_element_type=jnp.float32) + sw_ref[0, _N]

    logits = _dgt(xg, gw_ref[...]) + gb_ref[...]

    ei = jax.lax.broadcasted_iota(jnp.int32, (_B, _E), 1)
    m1 = jnp.max(logits, axis=1, keepdims=True)
    i1 = jnp.min(jnp.where(logits == m1, ei, _E), axis=1, keepdims=True)
    masked = jnp.where(ei == i1, -jnp.inf, logits)
    m2 = jnp.max(masked, axis=1, keepdims=True)
    i2 = jnp.min(jnp.where(masked == m2, ei, _E), axis=1, keepdims=True)
    p2 = jnp.exp(m2 - m1)
    zg = 1.0 + p2
    gates = jnp.where(ei == i1, 1.0 / zg, jnp.where(ei == i2, p2 / zg, 0.0))
    gates_ref[...] = gates

    imp = jnp.sum(gates, axis=0, keepdims=True)
    load = jnp.sum((gates > 0).astype(jnp.float32), axis=0, keepdims=True)

    def _cv(v):
        mu = jnp.sum(v, axis=1, keepdims=True) * (1.0 / _E)
        var = jnp.sum((v - mu) ** 2, axis=1, keepdims=True) * (1.0 / (_E - 1))
        return var / (mu * mu + 1e-10)

    loss_ref[...] = (_cv(imp) + _cv(load)) * 0.01


def _make_expert_kernel(p, pn, e):
    def _expert_kernel(g_ref, x_ref, ew_ref, eb_ref, wq_ref, bq_ref, wk_ref,
                       bk_ref, wv_ref, bv_ref, wo_ref, bo_ref, w1_ref, b1_ref,
                       w2_ref, b2_ref, pw_ref, pb_ref, out_ref):
        b = pl.program_id(0)
        g = g_ref[b, e]

        @pl.when(g != 0.0)
        def _compute():
            isq = 1.0 / np.sqrt(_DM)
            for n in range(_N):
                xn = x_ref[0, n]
                h = _dgt(xn, ew_ref[...]) + eb_ref[...]
                q = _dgt(h, wq_ref[...]) + bq_ref[...]
                k = _dgt(h, wk_ref[...]) + bk_ref[...]
                v = _dgt(h, wv_ref[...]) + bv_ref[...]
                s = _dgt(q, k) * isq
                sm = jnp.max(s, axis=1, keepdims=True)
                se = jnp.exp(s - sm)
                a = se / jnp.sum(se, axis=1, keepdims=True)
                o = jax.lax.dot_general(a, v, (((1,), (0,)), ((), ())),
                                        preferred_element_type=jnp.float32)
                h = h + _dgt(o, wo_ref[...]) + bo_ref[...]
                ff = jnp.maximum(_dgt(h, w1_ref[...]) + b1_ref[...], 0.0)
                h = h + _dgt(ff, w2_ref[...]) + b2_ref[...]
                out_ref[0, n] = g * (_dgt(h, pw_ref[...]) + pb_ref[...])

        @pl.when(g == 0.0)
        def _skip():
            out_ref[...] = jnp.zeros((1, _N, pn, p), jnp.float32)

    return _expert_kernel


def _run_expert(gates, x4, ep, p, e):
    pn = _L // p
    r2 = lambda a: a.reshape(1, -1)
    wlist = [ep['embed_w'], r2(ep['embed_b']), ep['wq'], r2(ep['bq']),
             ep['wk'], r2(ep['bk']), ep['wv'], r2(ep['bv']),
             ep['wo'], r2(ep['bo']), ep['w1'], r2(ep['b1']),
             ep['w2'], r2(ep['b2']), ep['proj_w'], r2(ep['proj_b'])]
    grid_spec = pltpu.PrefetchScalarGridSpec(
        num_scalar_prefetch=1,
        grid=(_B,),
        in_specs=[pl.BlockSpec((1, _N, pn, p), lambda b, g: (b, 0, 0, 0))]
        + [pl.BlockSpec(w.shape, lambda b, g: (0, 0)) for w in wlist],
        out_specs=pl.BlockSpec((1, _N, pn, p), lambda b, g: (b, 0, 0, 0)),
    )
    return pl.pallas_call(
        _make_expert_kernel(p, pn, e),
        grid_spec=grid_spec,
        out_shape=jax.ShapeDtypeStruct((_B, _N, pn, p), jnp.float32),
    )(gates, x4, *wlist)


def kernel(x, v_q, start_w, start_b, dec_w, dec_b, gate_w, gate_b, expert_params):
    x3 = x[..., 0]
    x_t = jnp.transpose(x3, (0, 2, 1))
    x2 = x_t.reshape(_B * _N, _L)

    dec = jnp.stack([dec_w[:, 0], dec_b])
    sw = jnp.concatenate([start_w[0], start_b]).reshape(1, _N + 1)

    gates, loss = pl.pallas_call(
        _gating_kernel,
        in_specs=[
            pl.BlockSpec(memory_space=pltpu.VMEM),
            pl.BlockSpec(memory_space=pltpu.SMEM),
            pl.BlockSpec(memory_space=pltpu.SMEM),
            pl.BlockSpec(memory_space=pltpu.VMEM),
            pl.BlockSpec(memory_space=pltpu.VMEM),
        ],
        out_shape=(jax.ShapeDtypeStruct((_B, _E), jnp.float32),
                   jax.ShapeDtypeStruct((1, 1), jnp.float32)),
    )(x2, dec, sw, gate_w, gate_b.reshape(1, _E))

    total = jnp.zeros((_B, _N, _L), jnp.float32)
    for e, p in enumerate(_PATCHES):
        pn = _L // p
        x4 = x_t.reshape(_B, _N, pn, p)
        eo = _run_expert(gates, x4, expert_params[e], p, e)
        total = total + eo.reshape(_B, _N, _L)

    out = x3 + jnp.transpose(total, (0, 2, 1))
    return out[..., None], loss[0, 0]


# speedup vs baseline: 2.4264x; 2.4264x over previous
import numpy as np
import jax
import jax.numpy as jnp
from jax.experimental import pallas as pl
from jax.experimental.pallas import tpu as pltpu

_B, _L, _N = 32, 336, 8
_E, _K = 4, 2
_DM, _DF = 128, 256
_PATCHES = (8, 6, 4, 2)
_MA_KERNELS = (4, 8, 12)


def _dgt(a, w):
    return jax.lax.dot_general(a, w, (((1,), (1,)), ((), ())),
                               preferred_element_type=jnp.float32)


def _gating_kernel(x2_ref, dec_ref, sw_ref, gw_ref, gb_ref, gates_ref, loss_ref):
    x = x2_ref[...]

    li = jax.lax.broadcasted_iota(jnp.int32, (_L, _L), 0)
    lp = jax.lax.broadcasted_iota(jnp.int32, (_L, _L), 1)
    means = []
    for k in _MA_KERNELS:
        lo = li - (k - 1) // 2
        hi = li + k // 2
        w = ((lp >= lo) & (lp <= hi)).astype(jnp.float32)
        w = w + jnp.where(lp == 0, jnp.maximum(-lo, 0).astype(jnp.float32), 0.0)
        w = w + jnp.where(lp == _L - 1,
                          jnp.maximum(hi - (_L - 1), 0).astype(jnp.float32), 0.0)
        wk = w * (1.0 / k)
        means.append(_dgt(x, wk))

    lgs = [x * dec_ref[0, k] + dec_ref[1, k] for k in range(len(_MA_KERNELS))]
    m = jnp.maximum(jnp.maximum(lgs[0], lgs[1]), lgs[2])
    es = [jnp.exp(lg - m) for lg in lgs]
    z = es[0] + es[1] + es[2]
    mean = (means[0] * es[0] + means[1] * es[1] + means[2] * es[2]) / z
    new_x = x + mean

    r = jax.lax.broadcasted_iota(jnp.int32, (_B, _B * _N), 0)
    c = jax.lax.broadcasted_iota(jnp.int32, (_B, _B * _N), 1)
    swcol = jnp.zeros((_B, _B * _N), jnp.float32)
    for n in range(_N):
        swcol = swcol + jnp.where((c & (_N - 1)) == n, sw_ref[0, n], 0.0)
    s_mat = jnp.where((c >> 3) == r, swcol, 0.0)
    xg = jax.lax.dot_general(s_mat, new_x, (((1,), (0,)), ((), ())),
                             preferred_element_type=jnp.float32) + sw_ref[0, _N]

    logits = _dgt(xg, gw_ref[...]) + gb_ref[...]

    ei = jax.lax.broadcasted_iota(jnp.int32, (_B, _E), 1)
    m1 = jnp.max(logits, axis=1, keepdims=True)
    i1 = jnp.min(jnp.where(logits == m1, ei, _E), axis=1, keepdims=True)
    masked = jnp.where(ei == i1, -jnp.inf, logits)
    m2 = jnp.max(masked, axis=1, keepdims=True)
    i2 = jnp.min(jnp.where(masked == m2, ei, _E), axis=1, keepdims=True)
    p2 = jnp.exp(m2 - m1)
    zg = 1.0 + p2
    gates = jnp.where(ei == i1, 1.0 / zg, jnp.where(ei == i2, p2 / zg, 0.0))
    gates_ref[...] = gates

    imp = jnp.sum(gates, axis=0, keepdims=True)
    load = jnp.sum((gates > 0).astype(jnp.float32), axis=0, keepdims=True)

    def _cv(v):
        mu = jnp.sum(v, axis=1, keepdims=True) * (1.0 / _E)
        var = jnp.sum((v - mu) ** 2, axis=1, keepdims=True) * (1.0 / (_E - 1))
        return var / (mu * mu + 1e-10)

    loss_ref[...] = (_cv(imp) + _cv(load)) * 0.01


def _make_expert_kernel(p, pn, pn8, e):
    def _expert_kernel(g_ref, x_ref, ew_ref, eb_ref, wq_ref, bq_ref, wk_ref,
                       bk_ref, wv_ref, bv_ref, wo_ref, bo_ref, w1_ref, b1_ref,
                       w2_ref, b2_ref, pw_ref, pb_ref, out_ref):
        b = pl.program_id(0)
        g = g_ref[b, e]

        @pl.when(g != 0.0)
        def _compute():
            isq = 1.0 / np.sqrt(_DM)
            xx = x_ref[0].reshape(_N * pn8, p)
            h = _dgt(xx, ew_ref[...]) + eb_ref[...]
            q = _dgt(h, wq_ref[...]) + bq_ref[...]
            k = _dgt(h, wk_ref[...]) + bk_ref[...]
            v = _dgt(h, wv_ref[...]) + bv_ref[...]
            q3 = q.reshape(_N, pn8, _DM)
            k3 = k.reshape(_N, pn8, _DM)
            v3 = v.reshape(_N, pn8, _DM)
            s = jax.lax.dot_general(q3, k3, (((2,), (2,)), ((0,), (0,))),
                                    preferred_element_type=jnp.float32) * isq
            if pn8 != pn:
                col = jax.lax.broadcasted_iota(jnp.int32, (_N, pn8, pn8), 2)
                s = jnp.where(col < pn, s, -jnp.inf)
            sm = jnp.max(s, axis=2, keepdims=True)
            se = jnp.exp(s - sm)
            a = se / jnp.sum(se, axis=2, keepdims=True)
            o3 = jax.lax.dot_general(a, v3, (((2,), (1,)), ((0,), (0,))),
                                     preferred_element_type=jnp.float32)
            o = o3.reshape(_N * pn8, _DM)
            h = h + _dgt(o, wo_ref[...]) + bo_ref[...]
            ff = jnp.maximum(_dgt(h, w1_ref[...]) + b1_ref[...], 0.0)
            h = h + _dgt(ff, w2_ref[...]) + b2_ref[...]
            oo = g * (_dgt(h, pw_ref[...]) + pb_ref[...])
            out_ref[...] = oo.reshape(1, _N, pn8, p)

        @pl.when(g == 0.0)
        def _skip():
            out_ref[...] = jnp.zeros((1, _N, pn8, p), jnp.float32)

    return _expert_kernel


def _run_expert(gates, x4p, ep, p, e):
    pn = _L // p
    pn8 = ((pn + 7) // 8) * 8
    r2 = lambda a: a.reshape(1, -1)
    wlist = [ep['embed_w'], r2(ep['embed_b']), ep['wq'], r2(ep['bq']),
             ep['wk'], r2(ep['bk']), ep['wv'], r2(ep['bv']),
             ep['wo'], r2(ep['bo']), ep['w1'], r2(ep['b1']),
             ep['w2'], r2(ep['b2']), ep['proj_w'], r2(ep['proj_b'])]
    grid_spec = pltpu.PrefetchScalarGridSpec(
        num_scalar_prefetch=1,
        grid=(_B,),
        in_specs=[pl.BlockSpec((1, _N, pn8, p), lambda b, g: (b, 0, 0, 0))]
        + [pl.BlockSpec(w.shape, lambda b, g: (0, 0)) for w in wlist],
        out_specs=pl.BlockSpec((1, _N, pn8, p), lambda b, g: (b, 0, 0, 0)),
    )
    return pl.pallas_call(
        _make_expert_kernel(p, pn, pn8, e),
        grid_spec=grid_spec,
        out_shape=jax.ShapeDtypeStruct((_B, _N, pn8, p), jnp.float32),
    )(gates, x4p, *wlist)


def kernel(x, v_q, start_w, start_b, dec_w, dec_b, gate_w, gate_b, expert_params):
    x3 = x[..., 0]
    x_t = jnp.transpose(x3, (0, 2, 1))
    x2 = x_t.reshape(_B * _N, _L)

    dec = jnp.stack([dec_w[:, 0], dec_b])
    sw = jnp.concatenate([start_w[0], start_b]).reshape(1, _N + 1)

    gates, loss = pl.pallas_call(
        _gating_kernel,
        in_specs=[
            pl.BlockSpec(memory_space=pltpu.VMEM),
            pl.BlockSpec(memory_space=pltpu.SMEM),
            pl.BlockSpec(memory_space=pltpu.SMEM),
            pl.BlockSpec(memory_space=pltpu.VMEM),
            pl.BlockSpec(memory_space=pltpu.VMEM),
        ],
        out_shape=(jax.ShapeDtypeStruct((_B, _E), jnp.float32),
                   jax.ShapeDtypeStruct((1, 1), jnp.float32)),
    )(x2, dec, sw, gate_w, gate_b.reshape(1, _E))

    total = jnp.zeros((_B, _N, _L), jnp.float32)
    for e, p in enumerate(_PATCHES):
        pn = _L // p
        pn8 = ((pn + 7) // 8) * 8
        x4 = x_t.reshape(_B, _N, pn, p)
        if pn8 != pn:
            x4 = jnp.pad(x4, ((0, 0), (0, 0), (0, pn8 - pn), (0, 0)))
        eo = _run_expert(gates, x4, expert_params[e], p, e)
        total = total + eo[:, :, :pn, :].reshape(_B, _N, _L)

    out = x3 + jnp.transpose(total, (0, 2, 1))
    return out[..., None], loss[0, 0]


# speedup vs baseline: 3.7377x; 1.5404x over previous
import numpy as np
import jax
import jax.numpy as jnp
from jax.experimental import pallas as pl
from jax.experimental.pallas import tpu as pltpu

_B, _L, _N = 32, 336, 8
_E, _K = 4, 2
_DM, _DF = 128, 256
_PATCHES = (8, 6, 4, 2)
_MA_KERNELS = (4, 8, 12)


def _dgt(a, w):
    return jax.lax.dot_general(a, w, (((1,), (1,)), ((), ())),
                               preferred_element_type=jnp.float32)


def _gating_kernel(x2_ref, dec_ref, sw_ref, gw_ref, gb_ref, gates_ref, loss_ref):
    x = x2_ref[...]

    li = jax.lax.broadcasted_iota(jnp.int32, (_L, _L), 0)
    lp = jax.lax.broadcasted_iota(jnp.int32, (_L, _L), 1)
    means = []
    for k in _MA_KERNELS:
        lo = li - (k - 1) // 2
        hi = li + k // 2
        w = ((lp >= lo) & (lp <= hi)).astype(jnp.float32)
        w = w + jnp.where(lp == 0, jnp.maximum(-lo, 0).astype(jnp.float32), 0.0)
        w = w + jnp.where(lp == _L - 1,
                          jnp.maximum(hi - (_L - 1), 0).astype(jnp.float32), 0.0)
        wk = w * (1.0 / k)
        means.append(_dgt(x, wk))

    lgs = [x * dec_ref[0, k] + dec_ref[1, k] for k in range(len(_MA_KERNELS))]
    m = jnp.maximum(jnp.maximum(lgs[0], lgs[1]), lgs[2])
    es = [jnp.exp(lg - m) for lg in lgs]
    z = es[0] + es[1] + es[2]
    mean = (means[0] * es[0] + means[1] * es[1] + means[2] * es[2]) / z
    new_x = x + mean

    r = jax.lax.broadcasted_iota(jnp.int32, (_B, _B * _N), 0)
    c = jax.lax.broadcasted_iota(jnp.int32, (_B, _B * _N), 1)
    swcol = jnp.zeros((_B, _B * _N), jnp.float32)
    for n in range(_N):
        swcol = swcol + jnp.where((c & (_N - 1)) == n, sw_ref[0, n], 0.0)
    s_mat = jnp.where((c >> 3) == r, swcol, 0.0)
    xg = jax.lax.dot_general(s_mat, new_x, (((1,), (0,)), ((), ())),
                             preferred_element_type=jnp.float32) + sw_ref[0, _N]

    logits = _dgt(xg, gw_ref[...]) + gb_ref[...]

    ei = jax.lax.broadcasted_iota(jnp.int32, (_B, _E), 1)
    m1 = jnp.max(logits, axis=1, keepdims=True)
    i1 = jnp.min(jnp.where(logits == m1, ei, _E), axis=1, keepdims=True)
    masked = jnp.where(ei == i1, -jnp.inf, logits)
    m2 = jnp.max(masked, axis=1, keepdims=True)
    i2 = jnp.min(jnp.where(masked == m2, ei, _E), axis=1, keepdims=True)
    p2 = jnp.exp(m2 - m1)
    zg = 1.0 + p2
    gates = jnp.where(ei == i1, 1.0 / zg, jnp.where(ei == i2, p2 / zg, 0.0))
    gates_ref[...] = gates

    imp = jnp.sum(gates, axis=0, keepdims=True)
    load = jnp.sum((gates > 0).astype(jnp.float32), axis=0, keepdims=True)

    def _cv(v):
        mu = jnp.sum(v, axis=1, keepdims=True) * (1.0 / _E)
        var = jnp.sum((v - mu) ** 2, axis=1, keepdims=True) * (1.0 / (_E - 1))
        return var / (mu * mu + 1e-10)

    loss_ref[...] = (_cv(imp) + _cv(load)) * 0.01


def _make_expert_kernel(p, pn, pn8, e):
    def _expert_kernel(g_ref, x_ref, ew_ref, eb_ref, wq_ref, bq_ref, wk_ref,
                       bk_ref, wv_ref, bv_ref, wo_ref, bo_ref, w1_ref, b1_ref,
                       w2_ref, b2_ref, pw_ref, pb_ref, out_ref):
        b = pl.program_id(0)
        g = g_ref[b, e]

        @pl.when(g != 0.0)
        def _compute():
            isq = 1.0 / np.sqrt(_DM)
            h = jnp.concatenate(
                [jax.lax.dot_general(x_ref[0, n], ew_ref[...],
                                     (((0,), (1,)), ((), ())),
                                     preferred_element_type=jnp.float32)
                 for n in range(_N)], axis=0) + eb_ref[...]
            q = _dgt(h, wq_ref[...]) + bq_ref[...]
            k = _dgt(h, wk_ref[...]) + bk_ref[...]
            v = _dgt(h, wv_ref[...]) + bv_ref[...]
            q3 = q.reshape(_N, pn8, _DM)
            k3 = k.reshape(_N, pn8, _DM)
            v3 = v.reshape(_N, pn8, _DM)
            s = jax.lax.dot_general(q3, k3, (((2,), (2,)), ((0,), (0,))),
                                    preferred_element_type=jnp.float32) * isq
            if pn8 != pn:
                col = jax.lax.broadcasted_iota(jnp.int32, (_N, pn8, pn8), 2)
                s = jnp.where(col < pn, s, -jnp.inf)
            sm = jnp.max(s, axis=2, keepdims=True)
            se = jnp.exp(s - sm)
            a = se / jnp.sum(se, axis=2, keepdims=True)
            o3 = jax.lax.dot_general(a, v3, (((2,), (1,)), ((0,), (0,))),
                                     preferred_element_type=jnp.float32)
            o = o3.reshape(_N * pn8, _DM)
            h = h + _dgt(o, wo_ref[...]) + bo_ref[...]
            ff = jnp.maximum(_dgt(h, w1_ref[...]) + b1_ref[...], 0.0)
            h = h + _dgt(ff, w2_ref[...]) + b2_ref[...]
            for n in range(_N):
                hn = h[n * pn8:(n + 1) * pn8, :]
                on = jax.lax.dot_general(pw_ref[...], hn,
                                         (((1,), (1,)), ((), ())),
                                         preferred_element_type=jnp.float32)
                out_ref[0, n] = g * (on + pb_ref[...])

        @pl.when(g == 0.0)
        def _skip():
            out_ref[...] = jnp.zeros((1, _N, p, pn8), jnp.float32)

    return _expert_kernel


def _run_expert(gates, x4p, ep, p, e):
    pn = _L // p
    pn8 = ((pn + 7) // 8) * 8
    r2 = lambda a: a.reshape(1, -1)
    wlist = [ep['embed_w'], r2(ep['embed_b']), ep['wq'], r2(ep['bq']),
             ep['wk'], r2(ep['bk']), ep['wv'], r2(ep['bv']),
             ep['wo'], r2(ep['bo']), ep['w1'], r2(ep['b1']),
             ep['w2'], r2(ep['b2']), ep['proj_w'], ep['proj_b'].reshape(-1, 1)]
    grid_spec = pltpu.PrefetchScalarGridSpec(
        num_scalar_prefetch=1,
        grid=(_B,),
        in_specs=[pl.BlockSpec((1, _N, p, pn8), lambda b, g: (b, 0, 0, 0))]
        + [pl.BlockSpec(w.shape, lambda b, g: (0, 0)) for w in wlist],
        out_specs=pl.BlockSpec((1, _N, p, pn8), lambda b, g: (b, 0, 0, 0)),
    )
    return pl.pallas_call(
        _make_expert_kernel(p, pn, pn8, e),
        grid_spec=grid_spec,
        out_shape=jax.ShapeDtypeStruct((_B, _N, p, pn8), jnp.float32),
    )(gates, x4p, *wlist)


def kernel(x, v_q, start_w, start_b, dec_w, dec_b, gate_w, gate_b, expert_params):
    x3 = x[..., 0]
    x_t = jnp.transpose(x3, (0, 2, 1))
    x2 = x_t.reshape(_B * _N, _L)

    dec = jnp.stack([dec_w[:, 0], dec_b])
    sw = jnp.concatenate([start_w[0], start_b]).reshape(1, _N + 1)

    gates, loss = pl.pallas_call(
        _gating_kernel,
        in_specs=[
            pl.BlockSpec(memory_space=pltpu.VMEM),
            pl.BlockSpec(memory_space=pltpu.SMEM),
            pl.BlockSpec(memory_space=pltpu.SMEM),
            pl.BlockSpec(memory_space=pltpu.VMEM),
            pl.BlockSpec(memory_space=pltpu.VMEM),
        ],
        out_shape=(jax.ShapeDtypeStruct((_B, _E), jnp.float32),
                   jax.ShapeDtypeStruct((1, 1), jnp.float32)),
    )(x2, dec, sw, gate_w, gate_b.reshape(1, _E))

    total = jnp.zeros((_B, _N, _L), jnp.float32)
    for e, p in enumerate(_PATCHES):
        pn = _L // p
        pn8 = ((pn + 7) // 8) * 8
        x4 = x_t.reshape(_B, _N, pn, p)
        if pn8 != pn:
            x4 = jnp.pad(x4, ((0, 0), (0, 0), (0, pn8 - pn), (0, 0)))
        x5 = jnp.transpose(x4, (0, 1, 3, 2))
        eo = _run_expert(gates, x5, expert_params[e], p, e)
        eo4 = jnp.transpose(eo, (0, 1, 3, 2))[:, :, :pn, :]
        total = total + eo4.reshape(_B, _N, _L)

    out = x3 + jnp.transpose(total, (0, 2, 1))
    return out[..., None], loss[0, 0]


# speedup vs baseline: 4.6856x; 1.2536x over previous
import numpy as np
import jax
import jax.numpy as jnp
from jax.experimental import pallas as pl
from jax.experimental.pallas import tpu as pltpu

_B, _L, _N = 32, 336, 8
_E, _K = 4, 2
_DM, _DF = 128, 256
_PATCHES = (8, 6, 4, 2)
_MA_KERNELS = (4, 8, 12)


def _dgt(a, w):
    return jax.lax.dot_general(a, w, (((1,), (1,)), ((), ())),
                               preferred_element_type=jnp.float32)


def _gating_kernel(x2_ref, dec_ref, sw_ref, gw_ref, gb_ref, gates_ref, loss_ref):
    x = x2_ref[...]

    li = jax.lax.broadcasted_iota(jnp.int32, (_L, _L), 0)
    lp = jax.lax.broadcasted_iota(jnp.int32, (_L, _L), 1)
    means = []
    for k in _MA_KERNELS:
        lo = li - (k - 1) // 2
        hi = li + k // 2
        w = ((lp >= lo) & (lp <= hi)).astype(jnp.float32)
        w = w + jnp.where(lp == 0, jnp.maximum(-lo, 0).astype(jnp.float32), 0.0)
        w = w + jnp.where(lp == _L - 1,
                          jnp.maximum(hi - (_L - 1), 0).astype(jnp.float32), 0.0)
        wk = w * (1.0 / k)
        means.append(_dgt(x, wk))

    lgs = [x * dec_ref[0, k] + dec_ref[1, k] for k in range(len(_MA_KERNELS))]
    m = jnp.maximum(jnp.maximum(lgs[0], lgs[1]), lgs[2])
    es = [jnp.exp(lg - m) for lg in lgs]
    z = es[0] + es[1] + es[2]
    mean = (means[0] * es[0] + means[1] * es[1] + means[2] * es[2]) / z
    new_x = x + mean

    r = jax.lax.broadcasted_iota(jnp.int32, (_B, _B * _N), 0)
    c = jax.lax.broadcasted_iota(jnp.int32, (_B, _B * _N), 1)
    swcol = jnp.zeros((_B, _B * _N), jnp.float32)
    for n in range(_N):
        swcol = swcol + jnp.where((c & (_N - 1)) == n, sw_ref[0, n], 0.0)
    s_mat = jnp.where((c >> 3) == r, swcol, 0.0)
    xg = jax.lax.dot_general(s_mat, new_x, (((1,), (0,)), ((), ())),
                             preferred_element_type=jnp.float32) + sw_ref[0, _N]

    logits = _dgt(xg, gw_ref[...]) + gb_ref[...]

    ei = jax.lax.broadcasted_iota(jnp.int32, (_B, _E), 1)
    m1 = jnp.max(logits, axis=1, keepdims=True)
    i1 = jnp.min(jnp.where(logits == m1, ei, _E), axis=1, keepdims=True)
    masked = jnp.where(ei == i1, -jnp.inf, logits)
    m2 = jnp.max(masked, axis=1, keepdims=True)
    i2 = jnp.min(jnp.where(masked == m2, ei, _E), axis=1, keepdims=True)
    p2 = jnp.exp(m2 - m1)
    zg = 1.0 + p2
    gates = jnp.where(ei == i1, 1.0 / zg, jnp.where(ei == i2, p2 / zg, 0.0))
    gates_ref[...] = gates

    imp = jnp.sum(gates, axis=0, keepdims=True)
    load = jnp.sum((gates > 0).astype(jnp.float32), axis=0, keepdims=True)

    def _cv(v):
        mu = jnp.sum(v, axis=1, keepdims=True) * (1.0 / _E)
        var = jnp.sum((v - mu) ** 2, axis=1, keepdims=True) * (1.0 / (_E - 1))
        return var / (mu * mu + 1e-10)

    loss_ref[...] = (_cv(imp) + _cv(load)) * 0.01


def _expert_body(g, p, pn, pn8, x_ref, wrefs, out_ref):
    (ew_ref, eb_ref, wq_ref, bq_ref, wk_ref, bk_ref, wv_ref, bv_ref,
     wo_ref, bo_ref, w1_ref, b1_ref, w2_ref, b2_ref, pw_ref, pb_ref) = wrefs

    @pl.when(g != 0.0)
    def _compute():
        isq = 1.0 / np.sqrt(_DM)
        h = jnp.concatenate(
            [jax.lax.dot_general(x_ref[0, n], ew_ref[...],
                                 (((0,), (1,)), ((), ())),
                                 preferred_element_type=jnp.float32)
             for n in range(_N)], axis=0) + eb_ref[...]
        q = _dgt(h, wq_ref[...]) + bq_ref[...]
        k = _dgt(h, wk_ref[...]) + bk_ref[...]
        v = _dgt(h, wv_ref[...]) + bv_ref[...]
        q3 = q.reshape(_N, pn8, _DM)
        k3 = k.reshape(_N, pn8, _DM)
        v3 = v.reshape(_N, pn8, _DM)
        s = jax.lax.dot_general(q3, k3, (((2,), (2,)), ((0,), (0,))),
                                preferred_element_type=jnp.float32) * isq
        if pn8 != pn:
            col = jax.lax.broadcasted_iota(jnp.int32, (_N, pn8, pn8), 2)
            s = jnp.where(col < pn, s, -jnp.inf)
        sm = jnp.max(s, axis=2, keepdims=True)
        se = jnp.exp(s - sm)
        a = se / jnp.sum(se, axis=2, keepdims=True)
        o3 = jax.lax.dot_general(a, v3, (((2,), (1,)), ((0,), (0,))),
                                 preferred_element_type=jnp.float32)
        o = o3.reshape(_N * pn8, _DM)
        h = h + _dgt(o, wo_ref[...]) + bo_ref[...]
        ff = jnp.maximum(_dgt(h, w1_ref[...]) + b1_ref[...], 0.0)
        h = h + _dgt(ff, w2_ref[...]) + b2_ref[...]
        for n in range(_N):
            hn = h[n * pn8:(n + 1) * pn8, :]
            on = jax.lax.dot_general(pw_ref[...], hn,
                                     (((1,), (1,)), ((), ())),
                                     preferred_element_type=jnp.float32)
            out_ref[0, n] = g * (on + pb_ref[...])

    @pl.when(g == 0.0)
    def _skip():
        out_ref[...] = jnp.zeros((1, _N, p, pn8), jnp.float32)


def _geom(p):
    pn = _L // p
    pn8 = ((pn + 7) // 8) * 8
    return pn, pn8


def _merged_expert_kernel(g_ref, *refs):
    b = pl.program_id(0)
    xs = refs[:4]
    outs = refs[-4:]
    for e, p in enumerate(_PATCHES):
        pn, pn8 = _geom(p)
        wrefs = refs[4 + 16 * e: 4 + 16 * (e + 1)]
        _expert_body(g_ref[b, e], p, pn, pn8, xs[e], wrefs, outs[e])


def _run_experts(gates, x5s, expert_params):
    r2 = lambda a: a.reshape(1, -1)
    wlist = []
    for ep in expert_params:
        wlist += [ep['embed_w'], r2(ep['embed_b']), ep['wq'], r2(ep['bq']),
                  ep['wk'], r2(ep['bk']), ep['wv'], r2(ep['bv']),
                  ep['wo'], r2(ep['bo']), ep['w1'], r2(ep['b1']),
                  ep['w2'], r2(ep['b2']), ep['proj_w'],
                  ep['proj_b'].reshape(-1, 1)]
    xspecs = [pl.BlockSpec((1, _N, p, _geom(p)[1]), lambda b, g: (b, 0, 0, 0))
              for p in _PATCHES]
    grid_spec = pltpu.PrefetchScalarGridSpec(
        num_scalar_prefetch=1,
        grid=(_B,),
        in_specs=xspecs + [pl.BlockSpec(w.shape, lambda b, g: (0, 0))
                           for w in wlist],
        out_specs=[pl.BlockSpec((1, _N, p, _geom(p)[1]),
                                lambda b, g: (b, 0, 0, 0)) for p in _PATCHES],
    )
    return pl.pallas_call(
        _merged_expert_kernel,
        grid_spec=grid_spec,
        out_shape=[jax.ShapeDtypeStruct((_B, _N, p, _geom(p)[1]), jnp.float32)
                   for p in _PATCHES],
    )(gates, *x5s, *wlist)


def kernel(x, v_q, start_w, start_b, dec_w, dec_b, gate_w, gate_b, expert_params):
    x3 = x[..., 0]
    x_t = jnp.transpose(x3, (0, 2, 1))
    x2 = x_t.reshape(_B * _N, _L)

    dec = jnp.stack([dec_w[:, 0], dec_b])
    sw = jnp.concatenate([start_w[0], start_b]).reshape(1, _N + 1)

    gates, loss = pl.pallas_call(
        _gating_kernel,
        in_specs=[
            pl.BlockSpec(memory_space=pltpu.VMEM),
            pl.BlockSpec(memory_space=pltpu.SMEM),
            pl.BlockSpec(memory_space=pltpu.SMEM),
            pl.BlockSpec(memory_space=pltpu.VMEM),
            pl.BlockSpec(memory_space=pltpu.VMEM),
        ],
        out_shape=(jax.ShapeDtypeStruct((_B, _E), jnp.float32),
                   jax.ShapeDtypeStruct((1, 1), jnp.float32)),
    )(x2, dec, sw, gate_w, gate_b.reshape(1, _E))

    x5s = []
    for p in _PATCHES:
        pn, pn8 = _geom(p)
        x4 = x_t.reshape(_B, _N, pn, p)
        if pn8 != pn:
            x4 = jnp.pad(x4, ((0, 0), (0, 0), (0, pn8 - pn), (0, 0)))
        x5s.append(jnp.transpose(x4, (0, 1, 3, 2)))

    eos = _run_experts(gates, x5s, expert_params)

    total = jnp.zeros((_B, _N, _L), jnp.float32)
    for e, p in enumerate(_PATCHES):
        pn, _ = _geom(p)
        eo4 = jnp.transpose(eos[e], (0, 1, 3, 2))[:, :, :pn, :]
        total = total + eo4.reshape(_B, _N, _L)

    out = x3 + jnp.transpose(total, (0, 2, 1))
    return out[..., None], loss[0, 0]
